# Initial kernel scaffold; baseline (speedup 1.0000x reference)
import jax
import jax.numpy as jnp
from jax.experimental import pallas as pl


def kernel(x_paper, x_author, batch_paper, batch_author):
    raise NotImplementedError("write your pallas kernel here")


# speedup vs baseline: 3.1374x; 3.1374x over previous
import functools

import jax
import jax.numpy as jnp
from jax import lax
from jax.experimental import pallas as pl
from jax.experimental.pallas import tpu as pltpu
from jax.experimental.pallas import tpu_sc as plsc

N = 320000
D = 128
G = 1024

_NS = 16
CHUNK = 128
ROWS_PER_TILE = N // _NS
NFULL = ROWS_PER_TILE // CHUNK
TAIL = ROWS_PER_TILE - NFULL * CHUNK
GSTRIPE = G // _NS


def _sc_segment_sums(x_paper, x_author, batch_paper, batch_author):
    mesh = plsc.VectorSubcoreMesh(core_axis_name="c", subcore_axis_name="s")

    @functools.partial(
        pl.kernel,
        mesh=mesh,
        out_type=[
            jax.ShapeDtypeStruct((2 * G, D), jnp.float32),
            jax.ShapeDtypeStruct((2 * G, D), jnp.float32),
        ],
        scratch_types=[
            pltpu.VMEM((CHUNK, D), jnp.float32),
            pltpu.VMEM((CHUNK,), jnp.int32),
            pltpu.VMEM((TAIL, D), jnp.float32),
            pltpu.VMEM((TAIL,), jnp.int32),
            pltpu.VMEM((CHUNK, D), jnp.float32),
            pltpu.VMEM_SHARED((G, D), jnp.float32),
            pltpu.VMEM_SHARED((G, D), jnp.float32),
        ],
    )
    def k(xp, xa, bp, ba, zeros_gd, ones_hbm,
          sums_out, cnts_out,
          row_buf, idx_buf, tail_buf, tail_idx, ones_buf, acc, cnt):
        cid = lax.axis_index("c")
        sid = lax.axis_index("s")

        pltpu.sync_copy(zeros_gd.at[pl.ds(sid * GSTRIPE, GSTRIPE)],
                        acc.at[pl.ds(sid * GSTRIPE, GSTRIPE)])
        pltpu.sync_copy(zeros_gd.at[pl.ds(sid * GSTRIPE, GSTRIPE)],
                        cnt.at[pl.ds(sid * GSTRIPE, GSTRIPE)])
        pltpu.sync_copy(ones_hbm, ones_buf)
        plsc.subcore_barrier()

        def process(x_hbm, b_hbm):
            tile_base = sid * ROWS_PER_TILE

            def body(i, carry):
                base = tile_base + i * CHUNK
                pltpu.sync_copy(b_hbm.at[pl.ds(base, CHUNK)], idx_buf)
                pltpu.sync_copy(x_hbm.at[pl.ds(base, CHUNK)], row_buf)
                pltpu.sync_copy(row_buf, acc.at[idx_buf], add=True)
                pltpu.sync_copy(ones_buf, cnt.at[idx_buf], add=True)
                return carry

            lax.fori_loop(0, NFULL, body, 0)
            base = tile_base + NFULL * CHUNK
            pltpu.sync_copy(b_hbm.at[pl.ds(base, TAIL)], tail_idx)
            pltpu.sync_copy(x_hbm.at[pl.ds(base, TAIL)], tail_buf)
            pltpu.sync_copy(tail_buf, acc.at[tail_idx], add=True)
            pltpu.sync_copy(ones_buf.at[pl.ds(0, TAIL)], cnt.at[tail_idx], add=True)

        pl.when(cid == 0)(lambda: process(xp, bp))
        pl.when(cid == 1)(lambda: process(xa, ba))

        plsc.subcore_barrier()
        off = cid * G + sid * GSTRIPE
        pltpu.sync_copy(acc.at[pl.ds(sid * GSTRIPE, GSTRIPE)],
                        sums_out.at[pl.ds(off, GSTRIPE)])
        pltpu.sync_copy(cnt.at[pl.ds(sid * GSTRIPE, GSTRIPE)],
                        cnts_out.at[pl.ds(off, GSTRIPE)])

    zeros_gd = jnp.zeros((G, D), jnp.float32)
    ones = jnp.ones((CHUNK, D), jnp.float32)
    return k(x_paper, x_author, batch_paper, batch_author, zeros_gd, ones)


def _finalize(sums, cnts):
    def body(s_ref, c_ref, o_ref):
        cp = jnp.maximum(c_ref[:G, 0:1], 1.0)
        ca = jnp.maximum(c_ref[G:, 0:1], 1.0)
        o_ref[...] = s_ref[:G] / cp + s_ref[G:] / ca

    return pl.pallas_call(
        body,
        out_shape=jax.ShapeDtypeStruct((G, D), jnp.float32),
    )(sums, cnts)


def kernel(x_paper, x_author, batch_paper, batch_author):
    sums, cnts = _sc_segment_sums(x_paper, x_author, batch_paper, batch_author)
    return _finalize(sums, cnts)


# speedup vs baseline: 4.2657x; 1.3597x over previous
import functools

import jax
import jax.numpy as jnp
from jax import lax
from jax.experimental import pallas as pl
from jax.experimental.pallas import tpu as pltpu
from jax.experimental.pallas import tpu_sc as plsc

N = 320000
D = 128
G = 1024

_NS = 16
CHUNK = 128
NCHUNKS = N // CHUNK
CPT = NCHUNKS // _NS
NEXTRA = NCHUNKS - CPT * _NS
NPAIRS = CPT // 2
IDXROWS = CPT + 12
NCPAD = 2512
GSTRIPE = G // _NS


def _sc_segment_sums(x_paper, x_author, batch_paper, batch_author):
    mesh = plsc.VectorSubcoreMesh(core_axis_name="c", subcore_axis_name="s")

    @functools.partial(
        pl.kernel,
        mesh=mesh,
        out_type=[
            jax.ShapeDtypeStruct((2 * G, D), jnp.float32),
            jax.ShapeDtypeStruct((2 * G, D), jnp.float32),
        ],
        scratch_types=[
            pltpu.VMEM((CHUNK, D), jnp.float32),
            pltpu.VMEM((CHUNK, D), jnp.float32),
            pltpu.VMEM((IDXROWS, CHUNK), jnp.int32),
            pltpu.VMEM((CHUNK, D), jnp.float32),
            pltpu.VMEM_SHARED((G, D), jnp.float32),
            pltpu.VMEM_SHARED((G, D), jnp.float32),
            pltpu.SemaphoreType.DMA,
            pltpu.SemaphoreType.DMA,
            pltpu.SemaphoreType.DMA,
            pltpu.SemaphoreType.DMA,
            pltpu.SemaphoreType.DMA,
            pltpu.SemaphoreType.DMA,
        ],
    )
    def k(xp, xa, bp2d, ba2d, zeros_gd, ones_hbm,
          sums_out, cnts_out,
          row0, row1, idx_blk, ones_buf, acc, cnt,
          l0, l1, s0, s1, o0, o1):
        cid = lax.axis_index("c")
        sid = lax.axis_index("s")

        pltpu.sync_copy(zeros_gd.at[pl.ds(sid * GSTRIPE, GSTRIPE)],
                        acc.at[pl.ds(sid * GSTRIPE, GSTRIPE)])
        pltpu.sync_copy(zeros_gd.at[pl.ds(sid * GSTRIPE, GSTRIPE)],
                        cnt.at[pl.ds(sid * GSTRIPE, GSTRIPE)])
        pltpu.sync_copy(ones_hbm, ones_buf)
        plsc.subcore_barrier()

        def process(x_hbm, b2d):
            extra_before = jnp.maximum(sid - (_NS - NEXTRA), 0)
            chunk0 = sid * CPT + extra_before
            row_base = chunk0 * CHUNK
            a0 = (chunk0 // 8) * 8
            doff = chunk0 - a0

            pltpu.sync_copy(b2d.at[pl.ds(a0, IDXROWS)], idx_blk)

            def load(c_local, buf, sem):
                pltpu.async_copy(
                    x_hbm.at[pl.ds(row_base + c_local * CHUNK, CHUNK)], buf, sem)

            def scat(c_local, buf, sem_s, sem_o):
                pltpu.async_copy(buf, acc.at[idx_blk.at[doff + c_local]], sem_s,
                                 add=True)
                pltpu.async_copy(ones_buf, cnt.at[idx_blk.at[doff + c_local]],
                                 sem_o, add=True)

            def wait_load(buf, sem):
                pltpu.make_async_copy(x_hbm.at[pl.ds(0, CHUNK)], buf, sem).wait()

            def wait_scat(buf, sem_s, sem_o):
                pltpu.make_async_copy(buf, acc.at[idx_blk.at[0]], sem_s).wait()
                pltpu.make_async_copy(ones_buf, cnt.at[idx_blk.at[0]],
                                      sem_o).wait()

            load(0, row0, l0)

            def body(p, carry):
                c0 = 2 * p
                c1 = 2 * p + 1
                wait_load(row0, l0)
                scat(c0, row0, s0, o0)
                pl.when(p > 0)(lambda: wait_scat(row1, s1, o1))
                load(c1, row1, l1)
                wait_load(row1, l1)
                scat(c1, row1, s1, o1)
                wait_scat(row0, s0, o0)
                pl.when(p < NPAIRS - 1)(lambda: load(c0 + 2, row0, l0))
                return carry

            lax.fori_loop(0, NPAIRS, body, 0)
            wait_scat(row1, s1, o1)

            @pl.when(sid >= _NS - NEXTRA)
            def _():
                pltpu.sync_copy(x_hbm.at[pl.ds(row_base + CPT * CHUNK, CHUNK)],
                                row0)
                pltpu.sync_copy(row0, acc.at[idx_blk.at[doff + CPT]], add=True)
                pltpu.sync_copy(ones_buf, cnt.at[idx_blk.at[doff + CPT]],
                                add=True)

        pl.when(cid == 0)(lambda: process(xp, bp2d))
        pl.when(cid == 1)(lambda: process(xa, ba2d))

        plsc.subcore_barrier()
        off = cid * G + sid * GSTRIPE
        pltpu.sync_copy(acc.at[pl.ds(sid * GSTRIPE, GSTRIPE)],
                        sums_out.at[pl.ds(off, GSTRIPE)])
        pltpu.sync_copy(cnt.at[pl.ds(sid * GSTRIPE, GSTRIPE)],
                        cnts_out.at[pl.ds(off, GSTRIPE)])

    def _b2d(b):
        b2 = b.reshape(NCHUNKS, CHUNK)
        return jnp.concatenate(
            [b2, jnp.zeros((NCPAD - NCHUNKS, CHUNK), jnp.int32)], axis=0)

    zeros_gd = jnp.zeros((G, D), jnp.float32)
    ones = jnp.ones((CHUNK, D), jnp.float32)
    return k(x_paper, x_author, _b2d(batch_paper), _b2d(batch_author),
             zeros_gd, ones)


def _finalize(sums, cnts):
    def body(s_ref, c_ref, o_ref):
        cp = jnp.maximum(c_ref[:G, 0:1], 1.0)
        ca = jnp.maximum(c_ref[G:, 0:1], 1.0)
        o_ref[...] = s_ref[:G] / cp + s_ref[G:] / ca

    return pl.pallas_call(
        body,
        out_shape=jax.ShapeDtypeStruct((G, D), jnp.float32),
    )(sums, cnts)


def kernel(x_paper, x_author, batch_paper, batch_author):
    sums, cnts = _sc_segment_sums(x_paper, x_author, batch_paper, batch_author)
    return _finalize(sums, cnts)


# speedup vs baseline: 6.3649x; 1.4921x over previous
import functools

import jax
import jax.numpy as jnp
from jax import lax
from jax.experimental import pallas as pl
from jax.experimental.pallas import tpu as pltpu
from jax.experimental.pallas import tpu_sc as plsc

N = 320000
D = 128
G = 1024

_NS = 16
_L = 16
CHUNK = 128
NCHUNKS = N // CHUNK
CPT = NCHUNKS // _NS
NEXTRA = NCHUNKS - CPT * _NS
NPAIRS = CPT // 2
IDXROWS = CPT + 12
NCPAD = 2512
GSTRIPE = G // _NS


def _sc_segment_sums(x_paper, x_author, batch_paper, batch_author):
    mesh = plsc.VectorSubcoreMesh(core_axis_name="c", subcore_axis_name="s")

    @functools.partial(
        pl.kernel,
        mesh=mesh,
        out_type=[
            jax.ShapeDtypeStruct((2 * G, D), jnp.float32),
            jax.ShapeDtypeStruct((2 * G,), jnp.float32),
        ],
        scratch_types=[
            pltpu.VMEM((CHUNK, D), jnp.float32),
            pltpu.VMEM((CHUNK, D), jnp.float32),
            pltpu.VMEM((IDXROWS, CHUNK), jnp.int32),
            pltpu.VMEM((G,), jnp.float32),
            pltpu.VMEM((_NS, GSTRIPE), jnp.float32),
            pltpu.VMEM((GSTRIPE,), jnp.float32),
            pltpu.VMEM_SHARED((G, D), jnp.float32),
            pltpu.VMEM_SHARED((_NS, G), jnp.float32),
            pltpu.SemaphoreType.DMA,
            pltpu.SemaphoreType.DMA,
            pltpu.SemaphoreType.DMA,
            pltpu.SemaphoreType.DMA,
        ],
        compiler_params=pltpu.CompilerParams(needs_layout_passes=False),
    )
    def k(xp, xa, bp2d, ba2d, zeros_gd,
          sums_out, cnts_out,
          row0, row1, idx_blk, hist, stripe, cstripe, acc, hists,
          l0, l1, s0, s1):
        cid = lax.axis_index("c")
        sid = lax.axis_index("s")

        pltpu.sync_copy(zeros_gd.at[pl.ds(sid * GSTRIPE, GSTRIPE)],
                        acc.at[pl.ds(sid * GSTRIPE, GSTRIPE)])
        zero16 = jnp.zeros((_L,), jnp.float32)
        for i in range(G // _L):
            hist[pl.ds(i * _L, _L)] = zero16
        plsc.subcore_barrier()

        one16 = jnp.ones((_L,), jnp.float32)

        def process(x_hbm, b2d):
            extra_before = jnp.maximum(sid - (_NS - NEXTRA), 0)
            chunk0 = sid * CPT + extra_before
            row_base = chunk0 * CHUNK
            a0 = (chunk0 // 8) * 8
            doff = chunk0 - a0

            pltpu.sync_copy(b2d.at[pl.ds(a0, IDXROWS)], idx_blk)

            def load(c_local, buf, sem):
                pltpu.async_copy(
                    x_hbm.at[pl.ds(row_base + c_local * CHUNK, CHUNK)], buf, sem)

            def scat(c_local, buf, sem):
                pltpu.async_copy(buf, acc.at[idx_blk.at[doff + c_local]], sem,
                                 add=True)

            def wait_load(buf, sem):
                pltpu.make_async_copy(x_hbm.at[pl.ds(0, CHUNK)], buf, sem).wait()

            def wait_scat(buf, sem):
                pltpu.make_async_copy(buf, acc.at[idx_blk.at[0]], sem).wait()

            def count(c_local):
                for j in range(CHUNK // _L):
                    idx16 = idx_blk[doff + c_local, pl.ds(j * _L, _L)]
                    plsc.addupdate_scatter(hist, [idx16], one16)

            load(0, row0, l0)

            def body(p, carry):
                c0 = 2 * p
                c1 = 2 * p + 1
                wait_load(row0, l0)
                scat(c0, row0, s0)
                count(c0)
                pl.when(p > 0)(lambda: wait_scat(row1, s1))
                load(c1, row1, l1)
                wait_load(row1, l1)
                scat(c1, row1, s1)
                count(c1)
                wait_scat(row0, s0)
                pl.when(p < NPAIRS - 1)(lambda: load(c0 + 2, row0, l0))
                return carry

            lax.fori_loop(0, NPAIRS, body, 0)
            wait_scat(row1, s1)

            @pl.when(sid >= _NS - NEXTRA)
            def _():
                pltpu.sync_copy(x_hbm.at[pl.ds(row_base + CPT * CHUNK, CHUNK)],
                                row0)
                pltpu.sync_copy(row0, acc.at[idx_blk.at[doff + CPT]], add=True)
                count(CPT)

        pl.when(cid == 0)(lambda: process(xp, bp2d))
        pl.when(cid == 1)(lambda: process(xa, ba2d))

        pltpu.sync_copy(hist, hists.at[sid])
        plsc.subcore_barrier()
        for i in range(_NS):
            pltpu.sync_copy(hists.at[i, pl.ds(sid * GSTRIPE, GSTRIPE)],
                            stripe.at[i])
        for j in range(GSTRIPE // _L):
            tot = stripe[0, pl.ds(j * _L, _L)]
            for i in range(1, _NS):
                tot = tot + stripe[i, pl.ds(j * _L, _L)]
            cstripe[pl.ds(j * _L, _L)] = tot

        off = cid * G + sid * GSTRIPE
        pltpu.sync_copy(acc.at[pl.ds(sid * GSTRIPE, GSTRIPE)],
                        sums_out.at[pl.ds(off, GSTRIPE)])
        pltpu.sync_copy(cstripe, cnts_out.at[pl.ds(off, GSTRIPE)])

    def _b2d(b):
        b2 = b.reshape(NCHUNKS, CHUNK)
        return jnp.concatenate(
            [b2, jnp.zeros((NCPAD - NCHUNKS, CHUNK), jnp.int32)], axis=0)

    zeros_gd = jnp.zeros((G, D), jnp.float32)
    return k(x_paper, x_author, _b2d(batch_paper), _b2d(batch_author), zeros_gd)


def _finalize(sums, cnts):
    def body(s_ref, c_ref, o_ref):
        cp = jnp.maximum(c_ref[0], 1.0).reshape(G, 1)
        ca = jnp.maximum(c_ref[1], 1.0).reshape(G, 1)
        o_ref[...] = s_ref[:G] / cp + s_ref[G:] / ca

    return pl.pallas_call(
        body,
        out_shape=jax.ShapeDtypeStruct((G, D), jnp.float32),
    )(sums, cnts.reshape(2, G))


def kernel(x_paper, x_author, batch_paper, batch_author):
    sums, cnts = _sc_segment_sums(x_paper, x_author, batch_paper, batch_author)
    return _finalize(sums, cnts)


# speedup vs baseline: 6.8816x; 1.0812x over previous
import functools

import jax
import jax.numpy as jnp
from jax import lax
from jax.experimental import pallas as pl
from jax.experimental.pallas import tpu as pltpu
from jax.experimental.pallas import tpu_sc as plsc

N = 320000
D = 128
G = 1024

_NS = 16
_L = 16
CHUNK = 128
NCHUNKS = N // CHUNK
CPT = NCHUNKS // _NS
NEXTRA = NCHUNKS - CPT * _NS
NQUADS = CPT // 4
IDXROWS = CPT + 12
NCPAD = 2512
GSTRIPE = G // _NS


def _sc_segment_sums(x_paper, x_author, batch_paper, batch_author):
    mesh = plsc.VectorSubcoreMesh(core_axis_name="c", subcore_axis_name="s")

    @functools.partial(
        pl.kernel,
        mesh=mesh,
        out_type=[
            jax.ShapeDtypeStruct((2 * G, D), jnp.float32),
            jax.ShapeDtypeStruct((2 * G,), jnp.float32),
        ],
        scratch_types=[
            pltpu.VMEM((CHUNK, D), jnp.float32),
            pltpu.VMEM((CHUNK, D), jnp.float32),
            pltpu.VMEM((CHUNK, D), jnp.float32),
            pltpu.VMEM((CHUNK, D), jnp.float32),
            pltpu.VMEM((IDXROWS, CHUNK), jnp.int32),
            pltpu.VMEM((G,), jnp.float32),
            pltpu.VMEM((_NS, GSTRIPE), jnp.float32),
            pltpu.VMEM((GSTRIPE,), jnp.float32),
            pltpu.VMEM_SHARED((G, D), jnp.float32),
            pltpu.VMEM_SHARED((_NS, G), jnp.float32),
            pltpu.SemaphoreType.DMA,
            pltpu.SemaphoreType.DMA,
            pltpu.SemaphoreType.DMA,
            pltpu.SemaphoreType.DMA,
            pltpu.SemaphoreType.DMA,
            pltpu.SemaphoreType.DMA,
            pltpu.SemaphoreType.DMA,
            pltpu.SemaphoreType.DMA,
        ],
        compiler_params=pltpu.CompilerParams(needs_layout_passes=False),
    )
    def k(xp, xa, bp2d, ba2d, zeros_gd,
          sums_out, cnts_out,
          row0, row1, row2, row3, idx_blk, hist, stripe, cstripe, acc, hists,
          l0, l1, l2, l3, s0, s1, s2, s3):
        cid = lax.axis_index("c")
        sid = lax.axis_index("s")

        pltpu.sync_copy(zeros_gd.at[pl.ds(sid * GSTRIPE, GSTRIPE)],
                        acc.at[pl.ds(sid * GSTRIPE, GSTRIPE)])
        zero16 = jnp.zeros((_L,), jnp.float32)
        for i in range(G // _L):
            hist[pl.ds(i * _L, _L)] = zero16
        plsc.subcore_barrier()

        one16 = jnp.ones((_L,), jnp.float32)

        def process(x_hbm, b2d):
            extra_before = jnp.maximum(sid - (_NS - NEXTRA), 0)
            chunk0 = sid * CPT + extra_before
            row_base = chunk0 * CHUNK
            a0 = (chunk0 // 8) * 8
            doff = chunk0 - a0

            pltpu.sync_copy(b2d.at[pl.ds(a0, IDXROWS)], idx_blk)

            def load(c_local, buf, sem):
                pltpu.async_copy(
                    x_hbm.at[pl.ds(row_base + c_local * CHUNK, CHUNK)], buf, sem)

            def scat(c_local, buf, sem):
                pltpu.async_copy(buf, acc.at[idx_blk.at[doff + c_local]], sem,
                                 add=True)

            def wait_load(buf, sem):
                pltpu.make_async_copy(x_hbm.at[pl.ds(0, CHUNK)], buf, sem).wait()

            def wait_scat(buf, sem):
                pltpu.make_async_copy(buf, acc.at[idx_blk.at[0]], sem).wait()

            def count(c_local):
                for j in range(CHUNK // _L):
                    idx16 = idx_blk[doff + c_local, pl.ds(j * _L, _L)]
                    plsc.addupdate_scatter(hist, [idx16], one16)

            rows = (row0, row1, row2, row3)
            ls = (l0, l1, l2, l3)
            ss = (s0, s1, s2, s3)

            load(0, row0, l0)
            load(1, row1, l1)
            load(2, row2, l2)

            def body(q, carry):
                for b in range(4):
                    c = 4 * q + b
                    bp = (b - 1) % 4
                    wait_load(rows[b], ls[b])
                    scat(c, rows[b], ss[b])
                    count(c)
                    if b == 0:
                        pl.when(q > 0)(
                            lambda bp=bp: wait_scat(rows[bp], ss[bp]))
                        load(c + 3, rows[bp], ls[bp])
                    else:
                        wait_scat(rows[bp], ss[bp])
                        pl.when(q < NQUADS - 1)(
                            lambda c=c, bp=bp: load(c + 3, rows[bp], ls[bp]))
                return carry

            lax.fori_loop(0, NQUADS, body, 0)
            wait_scat(row3, s3)

            @pl.when(sid >= _NS - NEXTRA)
            def _():
                pltpu.sync_copy(x_hbm.at[pl.ds(row_base + CPT * CHUNK, CHUNK)],
                                row0)
                pltpu.sync_copy(row0, acc.at[idx_blk.at[doff + CPT]], add=True)
                count(CPT)

        pl.when(cid == 0)(lambda: process(xp, bp2d))
        pl.when(cid == 1)(lambda: process(xa, ba2d))

        pltpu.sync_copy(hist, hists.at[sid])
        plsc.subcore_barrier()
        for i in range(_NS):
            pltpu.sync_copy(hists.at[i, pl.ds(sid * GSTRIPE, GSTRIPE)],
                            stripe.at[i])
        for j in range(GSTRIPE // _L):
            tot = stripe[0, pl.ds(j * _L, _L)]
            for i in range(1, _NS):
                tot = tot + stripe[i, pl.ds(j * _L, _L)]
            cstripe[pl.ds(j * _L, _L)] = tot

        off = cid * G + sid * GSTRIPE
        pltpu.sync_copy(acc.at[pl.ds(sid * GSTRIPE, GSTRIPE)],
                        sums_out.at[pl.ds(off, GSTRIPE)])
        pltpu.sync_copy(cstripe, cnts_out.at[pl.ds(off, GSTRIPE)])

    def _b2d(b):
        b2 = b.reshape(NCHUNKS, CHUNK)
        return jnp.concatenate(
            [b2, jnp.zeros((NCPAD - NCHUNKS, CHUNK), jnp.int32)], axis=0)

    zeros_gd = jnp.zeros((G, D), jnp.float32)
    return k(x_paper, x_author, _b2d(batch_paper), _b2d(batch_author), zeros_gd)


def _finalize(sums, cnts):
    def body(s_ref, c_ref, o_ref):
        cp = jnp.maximum(c_ref[0], 1.0).reshape(G, 1)
        ca = jnp.maximum(c_ref[1], 1.0).reshape(G, 1)
        o_ref[...] = s_ref[:G] / cp + s_ref[G:] / ca

    return pl.pallas_call(
        body,
        out_shape=jax.ShapeDtypeStruct((G, D), jnp.float32),
    )(sums, cnts.reshape(2, G))


def kernel(x_paper, x_author, batch_paper, batch_author):
    sums, cnts = _sc_segment_sums(x_paper, x_author, batch_paper, batch_author)
    return _finalize(sums, cnts)


# speedup vs baseline: 6.9706x; 1.0129x over previous
import functools

import jax
import jax.numpy as jnp
from jax import lax
from jax.experimental import pallas as pl
from jax.experimental.pallas import tpu as pltpu
from jax.experimental.pallas import tpu_sc as plsc

N = 320000
D = 128
G = 1024

_NS = 16
_L = 16
CHUNK = 128
NCHUNKS = N // CHUNK
CPT = NCHUNKS // _NS
NEXTRA = NCHUNKS - CPT * _NS
NBUF = 6
AHEAD = 3
NSEXT = CPT // NBUF
IDXROWS = CPT + 12
NCPAD = 2512
GSTRIPE = G // _NS


def _sc_segment_sums(x_paper, x_author, batch_paper, batch_author):
    mesh = plsc.VectorSubcoreMesh(core_axis_name="c", subcore_axis_name="s")

    @functools.partial(
        pl.kernel,
        mesh=mesh,
        out_type=[
            jax.ShapeDtypeStruct((2 * G, D), jnp.float32),
            jax.ShapeDtypeStruct((2, _NS, G), jnp.float32),
        ],
        scratch_types=(
            [pltpu.VMEM((CHUNK, D), jnp.float32)] * NBUF
            + [
                pltpu.VMEM((IDXROWS, CHUNK), jnp.int32),
                pltpu.VMEM((G,), jnp.float32),
                pltpu.VMEM_SHARED((G, D), jnp.float32),
            ]
            + [pltpu.SemaphoreType.DMA] * NBUF
            + [pltpu.SemaphoreType.DMA] * NBUF
        ),
        compiler_params=pltpu.CompilerParams(needs_layout_passes=False),
    )
    def k(xp, xa, bp2d, ba2d, zeros_gd, sums_out, cnts_out, *refs):
        rows = refs[:NBUF]
        idx_blk, hist, acc = refs[NBUF:NBUF + 3]
        ls = refs[NBUF + 3:2 * NBUF + 3]
        ss = refs[2 * NBUF + 3:]
        cid = lax.axis_index("c")
        sid = lax.axis_index("s")

        pltpu.sync_copy(zeros_gd.at[pl.ds(sid * GSTRIPE, GSTRIPE)],
                        acc.at[pl.ds(sid * GSTRIPE, GSTRIPE)])
        zero16 = jnp.zeros((_L,), jnp.float32)
        for i in range(G // _L):
            hist[pl.ds(i * _L, _L)] = zero16
        plsc.subcore_barrier()

        one16 = jnp.ones((_L,), jnp.float32)

        def process(x_hbm, b2d):
            extra_before = jnp.maximum(sid - (_NS - NEXTRA), 0)
            chunk0 = sid * CPT + extra_before
            row_base = chunk0 * CHUNK
            a0 = (chunk0 // 8) * 8
            doff = chunk0 - a0

            def load(c_local, buf, sem):
                pltpu.async_copy(
                    x_hbm.at[pl.ds(row_base + c_local * CHUNK, CHUNK)], buf, sem)

            def scat(c_local, buf, sem):
                pltpu.async_copy(buf, acc.at[idx_blk.at[doff + c_local]], sem,
                                 add=True)

            def wait_load(buf, sem):
                pltpu.make_async_copy(x_hbm.at[pl.ds(0, CHUNK)], buf, sem).wait()

            def wait_scat(buf, sem):
                pltpu.make_async_copy(buf, acc.at[idx_blk.at[0]], sem).wait()

            def count(c_local):
                for j in range(CHUNK // _L):
                    idx16 = idx_blk[doff + c_local, pl.ds(j * _L, _L)]
                    plsc.addupdate_scatter(hist, [idx16], one16)

            for b in range(AHEAD):
                load(b, rows[b], ls[b])
            pltpu.sync_copy(b2d.at[pl.ds(a0, IDXROWS)], idx_blk)

            def body(q, carry):
                for b in range(NBUF):
                    c = NBUF * q + b
                    wait_load(rows[b], ls[b])
                    scat(c, rows[b], ss[b])
                    count(c)
                    t = (b + AHEAD) % NBUF
                    if b < AHEAD:
                        pl.when(q > 0)(lambda t=t: wait_scat(rows[t], ss[t]))
                        load(c + AHEAD, rows[t], ls[t])
                    else:
                        wait_scat(rows[t], ss[t])
                        pl.when(q < NSEXT - 1)(
                            lambda c=c, t=t: load(c + AHEAD, rows[t], ls[t]))
                return carry

            lax.fori_loop(0, NSEXT, body, 0)
            for b in range(NBUF - AHEAD, NBUF):
                wait_scat(rows[b], ss[b])

            @pl.when(sid >= _NS - NEXTRA)
            def _():
                pltpu.sync_copy(x_hbm.at[pl.ds(row_base + CPT * CHUNK, CHUNK)],
                                rows[0])
                pltpu.sync_copy(rows[0], acc.at[idx_blk.at[doff + CPT]],
                                add=True)
                count(CPT)

        pl.when(cid == 0)(lambda: process(xp, bp2d))
        pl.when(cid == 1)(lambda: process(xa, ba2d))

        pltpu.sync_copy(hist, cnts_out.at[cid, sid])

        plsc.subcore_barrier()
        off = cid * G + sid * GSTRIPE
        pltpu.sync_copy(acc.at[pl.ds(sid * GSTRIPE, GSTRIPE)],
                        sums_out.at[pl.ds(off, GSTRIPE)])

    def _b2d(b):
        b2 = b.reshape(NCHUNKS, CHUNK)
        return jnp.concatenate(
            [b2, jnp.zeros((NCPAD - NCHUNKS, CHUNK), jnp.int32)], axis=0)

    zeros_gd = jnp.zeros((G, D), jnp.float32)
    return k(x_paper, x_author, _b2d(batch_paper), _b2d(batch_author), zeros_gd)


def _finalize(sums, cnts):
    def body(s_ref, c_ref, o_ref):
        cp = jnp.maximum(jnp.sum(c_ref[0], axis=0), 1.0).reshape(G, 1)
        ca = jnp.maximum(jnp.sum(c_ref[1], axis=0), 1.0).reshape(G, 1)
        o_ref[...] = s_ref[:G] / cp + s_ref[G:] / ca

    return pl.pallas_call(
        body,
        out_shape=jax.ShapeDtypeStruct((G, D), jnp.float32),
    )(sums, cnts)


def kernel(x_paper, x_author, batch_paper, batch_author):
    sums, cnts = _sc_segment_sums(x_paper, x_author, batch_paper, batch_author)
    return _finalize(sums, cnts)


# speedup vs baseline: 6.9774x; 1.0010x over previous
import functools

import jax
import jax.numpy as jnp
from jax import lax
from jax.experimental import pallas as pl
from jax.experimental.pallas import tpu as pltpu
from jax.experimental.pallas import tpu_sc as plsc

N = 320000
D = 128
G = 1024

_NS = 16
_L = 16
CHUNK = 128
NCHUNKS = N // CHUNK
CPT = NCHUNKS // _NS
NEXTRA = NCHUNKS - CPT * _NS
NBUF = 6
AHEAD = 2
NSEXT = CPT // NBUF
IDXROWS = CPT + 12
NCPAD = 2512
GSTRIPE = G // _NS


def _sc_segment_sums(x_paper, x_author, batch_paper, batch_author):
    mesh = plsc.VectorSubcoreMesh(core_axis_name="c", subcore_axis_name="s")

    @functools.partial(
        pl.kernel,
        mesh=mesh,
        out_type=[
            jax.ShapeDtypeStruct((2 * G, D), jnp.float32),
            jax.ShapeDtypeStruct((2, _NS, G), jnp.float32),
        ],
        scratch_types=(
            [pltpu.VMEM((CHUNK, D), jnp.float32)] * NBUF
            + [
                pltpu.VMEM((IDXROWS, CHUNK), jnp.int32),
                pltpu.VMEM((G,), jnp.float32),
                pltpu.VMEM_SHARED((G, D), jnp.float32),
            ]
            + [pltpu.SemaphoreType.DMA] * NBUF
            + [pltpu.SemaphoreType.DMA] * NBUF
        ),
        compiler_params=pltpu.CompilerParams(needs_layout_passes=False),
    )
    def k(xp, xa, bp2d, ba2d, zeros_gd, sums_out, cnts_out, *refs):
        rows = refs[:NBUF]
        idx_blk, hist, acc = refs[NBUF:NBUF + 3]
        ls = refs[NBUF + 3:2 * NBUF + 3]
        ss = refs[2 * NBUF + 3:]
        cid = lax.axis_index("c")
        sid = lax.axis_index("s")

        pltpu.sync_copy(zeros_gd.at[pl.ds(sid * GSTRIPE, GSTRIPE)],
                        acc.at[pl.ds(sid * GSTRIPE, GSTRIPE)])
        zero16 = jnp.zeros((_L,), jnp.float32)
        for i in range(G // _L):
            hist[pl.ds(i * _L, _L)] = zero16
        plsc.subcore_barrier()

        one16 = jnp.ones((_L,), jnp.float32)

        def process(x_hbm, b2d):
            extra_before = jnp.maximum(sid - (_NS - NEXTRA), 0)
            chunk0 = sid * CPT + extra_before
            row_base = chunk0 * CHUNK
            a0 = (chunk0 // 8) * 8
            doff = chunk0 - a0

            def load(c_local, buf, sem):
                pltpu.async_copy(
                    x_hbm.at[pl.ds(row_base + c_local * CHUNK, CHUNK)], buf, sem)

            def scat(c_local, buf, sem):
                pltpu.async_copy(buf, acc.at[idx_blk.at[doff + c_local]], sem,
                                 add=True)

            def wait_load(buf, sem):
                pltpu.make_async_copy(x_hbm.at[pl.ds(0, CHUNK)], buf, sem).wait()

            def wait_scat(buf, sem):
                pltpu.make_async_copy(buf, acc.at[idx_blk.at[0]], sem).wait()

            def count(c_local):
                for j in range(CHUNK // _L):
                    idx16 = idx_blk[doff + c_local, pl.ds(j * _L, _L)]
                    plsc.addupdate_scatter(hist, [idx16], one16)

            for b in range(AHEAD):
                load(b, rows[b], ls[b])
            pltpu.sync_copy(b2d.at[pl.ds(a0, IDXROWS)], idx_blk)

            def body(q, carry):
                for b in range(NBUF):
                    c = NBUF * q + b
                    wait_load(rows[b], ls[b])
                    scat(c, rows[b], ss[b])
                    count(c)
                    t = (b + AHEAD) % NBUF
                    if b < NBUF - AHEAD:
                        pl.when(q > 0)(lambda t=t: wait_scat(rows[t], ss[t]))
                    else:
                        wait_scat(rows[t], ss[t])
                    qmax = (CPT - 1 - AHEAD - b) // NBUF
                    if qmax >= NSEXT - 1:
                        load(c + AHEAD, rows[t], ls[t])
                    else:
                        pl.when(q <= qmax)(
                            lambda c=c, t=t: load(c + AHEAD, rows[t], ls[t]))
                return carry

            lax.fori_loop(0, NSEXT, body, 0)
            for b in range(AHEAD, NBUF):
                wait_scat(rows[b], ss[b])

            @pl.when(sid >= _NS - NEXTRA)
            def _():
                pltpu.sync_copy(x_hbm.at[pl.ds(row_base + CPT * CHUNK, CHUNK)],
                                rows[0])
                pltpu.sync_copy(rows[0], acc.at[idx_blk.at[doff + CPT]],
                                add=True)
                count(CPT)

        pl.when(cid == 0)(lambda: process(xp, bp2d))
        pl.when(cid == 1)(lambda: process(xa, ba2d))

        pltpu.sync_copy(hist, cnts_out.at[cid, sid])

        plsc.subcore_barrier()
        off = cid * G + sid * GSTRIPE
        pltpu.sync_copy(acc.at[pl.ds(sid * GSTRIPE, GSTRIPE)],
                        sums_out.at[pl.ds(off, GSTRIPE)])

    def _b2d(b):
        b2 = b.reshape(NCHUNKS, CHUNK)
        return jnp.concatenate(
            [b2, jnp.zeros((NCPAD - NCHUNKS, CHUNK), jnp.int32)], axis=0)

    zeros_gd = jnp.zeros((G, D), jnp.float32)
    return k(x_paper, x_author, _b2d(batch_paper), _b2d(batch_author), zeros_gd)


def _finalize(sums, cnts):
    def body(s_ref, c_ref, o_ref):
        cp = jnp.maximum(jnp.sum(c_ref[0], axis=0), 1.0).reshape(G, 1)
        ca = jnp.maximum(jnp.sum(c_ref[1], axis=0), 1.0).reshape(G, 1)
        o_ref[...] = s_ref[:G] / cp + s_ref[G:] / ca

    return pl.pallas_call(
        body,
        out_shape=jax.ShapeDtypeStruct((G, D), jnp.float32),
    )(sums, cnts)


def kernel(x_paper, x_author, batch_paper, batch_author):
    sums, cnts = _sc_segment_sums(x_paper, x_author, batch_paper, batch_author)
    return _finalize(sums, cnts)


# speedup vs baseline: 7.5880x; 1.0875x over previous
import functools

import jax
import jax.numpy as jnp
from jax import lax
from jax.experimental import pallas as pl
from jax.experimental.pallas import tpu as pltpu
from jax.experimental.pallas import tpu_sc as plsc

N = 320000
D = 128
G = 1024

_NS = 16
_L = 16
CHUNK = 128
NCHUNKS = N // CHUNK
CPT = NCHUNKS // _NS
NEXTRA = NCHUNKS - CPT * _NS
NBUF = 4
AHEAD = 2
NSEXT = CPT // NBUF
IDXROWS = CPT + 12
NCPAD = 2512
GSTRIPE = G // _NS


def _sc_segment_sums(x_paper, x_author, batch_paper, batch_author):
    mesh = plsc.VectorSubcoreMesh(core_axis_name="c", subcore_axis_name="s")

    @functools.partial(
        pl.kernel,
        mesh=mesh,
        out_type=[
            jax.ShapeDtypeStruct((2 * G, D), jnp.float32),
            jax.ShapeDtypeStruct((2, _NS, G), jnp.float32),
        ],
        scratch_types=(
            [pltpu.VMEM((CHUNK, D), jnp.float32)] * NBUF
            + [
                pltpu.VMEM((IDXROWS, CHUNK), jnp.int32),
                pltpu.VMEM((G,), jnp.float32),
                pltpu.VMEM((_L, D), jnp.float32),
                pltpu.VMEM((1, _L), jnp.int32),
                pltpu.VMEM_SHARED((G, D), jnp.float32),
            ]
            + [pltpu.SemaphoreType.DMA] * NBUF
            + [pltpu.SemaphoreType.DMA] * NBUF
            + [pltpu.SemaphoreType.DMA]
        ),
        compiler_params=pltpu.CompilerParams(needs_layout_passes=False),
    )
    def k(xp, xa, bp2d, ba2d, zeros_gd, sums_out, cnts_out, *refs):
        rows = refs[:NBUF]
        idx_blk, hist, flush_buf, seg_row, acc = refs[NBUF:NBUF + 5]
        ls = refs[NBUF + 5:2 * NBUF + 5]
        ss = refs[2 * NBUF + 5:3 * NBUF + 5]
        fs = refs[3 * NBUF + 5]
        cid = lax.axis_index("c")
        sid = lax.axis_index("s")

        pltpu.sync_copy(zeros_gd.at[pl.ds(sid * GSTRIPE, GSTRIPE)],
                        acc.at[pl.ds(sid * GSTRIPE, GSTRIPE)])
        zero16 = jnp.zeros((_L,), jnp.float32)
        for i in range(G // _L):
            hist[pl.ds(i * _L, _L)] = zero16
        plsc.subcore_barrier()

        one16 = jnp.ones((_L,), jnp.float32)

        def process(x_hbm, b2d):
            extra_before = jnp.maximum(sid - (_NS - NEXTRA), 0)
            chunk0 = sid * CPT + extra_before
            row_base = chunk0 * CHUNK
            a0 = (chunk0 // 8) * 8
            doff = chunk0 - a0

            def load(c_local, buf, sem):
                pltpu.async_copy(
                    x_hbm.at[pl.ds(row_base + c_local * CHUNK, CHUNK)], buf, sem)

            def scat(c_local, buf, sem):
                pltpu.async_copy(buf, acc.at[idx_blk.at[doff + c_local]], sem,
                                 add=True)

            def wait_load(buf, sem):
                pltpu.make_async_copy(x_hbm.at[pl.ds(0, CHUNK)], buf, sem).wait()

            def wait_scat(buf, sem):
                pltpu.make_async_copy(buf, acc.at[idx_blk.at[0]], sem).wait()

            def count(c_local):
                for j in range(CHUNK // _L):
                    idx16 = idx_blk[doff + c_local, pl.ds(j * _L, _L)]
                    plsc.addupdate_scatter(hist, [idx16], one16)

            def wait_flush():
                pltpu.make_async_copy(flush_buf, acc.at[seg_row.at[0]],
                                      fs).wait()

            def single_flush(b, first, ff):
                def tbody(r, sacc):
                    out = []
                    for j in range(D // _L):
                        s = sacc[j]
                        for rr in range(_L):
                            s = s + rows[b][r * _L + rr, pl.ds(j * _L, _L)]
                        out.append(s)
                    return tuple(out)

                zero = jnp.zeros((_L,), jnp.float32)
                sums8 = lax.fori_loop(0, CHUNK // _L, tbody, (zero,) * (D // _L))
                pl.when(ff > 0)(wait_flush)
                seg_row[0, pl.ds(0, _L)] = jnp.full((_L,), first, jnp.int32)
                for j in range(D // _L):
                    flush_buf[0, pl.ds(j * _L, _L)] = sums8[j]
                pltpu.async_copy(flush_buf, acc.at[seg_row.at[0]], fs, add=True)

            for b in range(AHEAD):
                load(b, rows[b], ls[b])
            pltpu.sync_copy(b2d.at[pl.ds(a0, IDXROWS)], idx_blk)
            pltpu.sync_copy(zeros_gd.at[pl.ds(0, _L)], flush_buf)

            def body(q, carry):
                ff = carry[0]
                fl = list(carry[1:])
                for b in range(NBUF):
                    c = NBUF * q + b
                    dc = doff + c
                    wait_load(rows[b], ls[b])
                    vf = idx_blk[dc, pl.ds(0, _L)]
                    vl = idx_blk[dc, pl.ds(CHUNK - _L, _L)]
                    first = jnp.min(vf)
                    issingle = first == jnp.max(vl)
                    count(c)
                    pl.when(jnp.logical_not(issingle))(
                        lambda b=b, c=c: scat(c, rows[b], ss[b]))
                    pl.when(issingle)(
                        lambda b=b, first=first, ff=ff: single_flush(
                            b, first, ff))
                    ff = jnp.where(issingle, 1, ff)
                    fl[b] = jnp.where(issingle, 0, 1)
                    t = (b + AHEAD) % NBUF
                    pl.when(fl[t] > 0)(lambda t=t: wait_scat(rows[t], ss[t]))
                    fl[t] = jnp.int32(0)
                    qmax = (CPT - 1 - AHEAD - b) // NBUF
                    if qmax >= NSEXT - 1:
                        load(c + AHEAD, rows[t], ls[t])
                    else:
                        pl.when(q <= qmax)(
                            lambda c=c, t=t: load(c + AHEAD, rows[t], ls[t]))
                return (ff, *fl)

            zero_flags = (jnp.int32(0),) * (NBUF + 1)
            final = lax.fori_loop(0, NSEXT, body, zero_flags)
            for b in range(NBUF):
                pl.when(final[1 + b] > 0)(
                    lambda b=b: wait_scat(rows[b], ss[b]))
            pl.when(final[0] > 0)(wait_flush)

            @pl.when(sid >= _NS - NEXTRA)
            def _():
                pltpu.sync_copy(x_hbm.at[pl.ds(row_base + CPT * CHUNK, CHUNK)],
                                rows[0])
                pltpu.sync_copy(rows[0], acc.at[idx_blk.at[doff + CPT]],
                                add=True)
                count(CPT)

        pl.when(cid == 0)(lambda: process(xp, bp2d))
        pl.when(cid == 1)(lambda: process(xa, ba2d))

        pltpu.sync_copy(hist, cnts_out.at[cid, sid])

        plsc.subcore_barrier()
        off = cid * G + sid * GSTRIPE
        pltpu.sync_copy(acc.at[pl.ds(sid * GSTRIPE, GSTRIPE)],
                        sums_out.at[pl.ds(off, GSTRIPE)])

    def _b2d(b):
        b2 = b.reshape(NCHUNKS, CHUNK)
        return jnp.concatenate(
            [b2, jnp.zeros((NCPAD - NCHUNKS, CHUNK), jnp.int32)], axis=0)

    zeros_gd = jnp.zeros((G, D), jnp.float32)
    return k(x_paper, x_author, _b2d(batch_paper), _b2d(batch_author), zeros_gd)


def _finalize(sums, cnts):
    def body(s_ref, c_ref, o_ref):
        cp = jnp.maximum(jnp.sum(c_ref[0], axis=0), 1.0).reshape(G, 1)
        ca = jnp.maximum(jnp.sum(c_ref[1], axis=0), 1.0).reshape(G, 1)
        o_ref[...] = s_ref[:G] / cp + s_ref[G:] / ca

    return pl.pallas_call(
        body,
        out_shape=jax.ShapeDtypeStruct((G, D), jnp.float32),
    )(sums, cnts)


def kernel(x_paper, x_author, batch_paper, batch_author):
    sums, cnts = _sc_segment_sums(x_paper, x_author, batch_paper, batch_author)
    return _finalize(sums, cnts)
